# baseline (speedup 1.0000x reference)
import functools

import jax
import jax.numpy as jnp
from jax import lax
from jax.experimental import pallas as pl
from jax.experimental.pallas import tpu as pltpu
from jax.experimental.pallas import tpu_sc as plsc

N = 10000
E = 320000
D_IN = 128
D_H = 64
D_OUT = 16
WEIGHT_DECAY = 0.0005

NPAD = 10240
C = 256
NCH = 40
ZB = 128
EPT = C * NCH
EPAD = 32 * EPT
ROWS_PER_TILE = NPAD // 16


def _spmm_sc(h, srcr, dstr, wr, d):
    mesh = plsc.VectorSubcoreMesh(core_axis_name="c", subcore_axis_name="s")

    @functools.partial(
        pl.kernel,
        out_type=jax.ShapeDtypeStruct((2, NPAD, d), jnp.float32),
        mesh=mesh,
        compiler_params=pltpu.CompilerParams(needs_layout_passes=False,
                                             use_tc_tiling_on_sc=False),
        scratch_types=[
            pltpu.VMEM((NCH, C), jnp.int32),
            pltpu.VMEM((NCH, C), jnp.int32),
            pltpu.VMEM((EPT,), jnp.float32),
            pltpu.VMEM((C, d), jnp.float32),
            pltpu.VMEM((C, d), jnp.float32),
            pltpu.VMEM((ZB, d), jnp.float32),
            pltpu.VMEM_SHARED((NPAD, d), jnp.float32),
            pltpu.SemaphoreType.DMA,
            pltpu.SemaphoreType.DMA,
        ],
    )
    def k(h_hbm, src_hbm, dst_hbm, w_hbm, out_hbm,
          srcv, dstv, wv, bufa, bufb, zbuf, acc, sema, semb):
        cid = lax.axis_index("c")
        sid = lax.axis_index("s")
        wid = cid * 16 + sid

        zero16 = jnp.zeros((16,), jnp.float32)
        iota0 = lax.iota(jnp.int32, 16)

        @pl.loop(0, ZB)
        def _(r):
            rv = jnp.full((16,), r, jnp.int32)
            for fb in range(d // 16):
                plsc.store_scatter(zbuf, [rv, iota0 + (fb * 16)], zero16)

        zbase = sid * ROWS_PER_TILE
        for i in range(ROWS_PER_TILE // ZB):
            pltpu.sync_copy(zbuf, acc.at[pl.ds(zbase + i * ZB, ZB)])

        pltpu.sync_copy(src_hbm.at[wid], srcv)
        pltpu.sync_copy(dst_hbm.at[wid], dstv)
        pltpu.sync_copy(w_hbm.at[wid], wv)

        plsc.subcore_barrier()

        dnums = lax.GatherDimensionNumbers(
            offset_dims=(), collapsed_slice_dims=(0,), start_index_map=(0,))
        idx16 = [jnp.full((16, 1), l, jnp.int32) for l in range(16)]

        def scale(buf, j):
            @pl.loop(0, C // 16)
            def _(g):
                wvec = wv.at[pl.ds(j * C + g * 16, 16)][...]
                for l in range(16):
                    s = lax.gather(
                        wvec, idx16[l], dnums, slice_sizes=(1,),
                        mode=lax.GatherScatterMode.PROMISE_IN_BOUNDS)
                    e = g * 16 + l
                    for fb in range(d // 16):
                        o = fb * 16
                        buf.at[e, pl.ds(o, 16)][...] = (
                            buf.at[e, pl.ds(o, 16)][...] * s)

        pltpu.async_copy(h_hbm.at[srcv.at[0]], bufa, sema)
        pltpu.async_copy(h_hbm.at[srcv.at[1]], bufb, semb)

        @pl.loop(0, NCH, step=2)
        def _(j):
            pltpu.make_async_copy(h_hbm.at[srcv.at[j]], bufa, sema).wait()
            scale(bufa, j)
            pltpu.sync_copy(bufa, acc.at[dstv.at[j]], add=True)

            @pl.when(j + 2 < NCH)
            def _():
                pltpu.async_copy(h_hbm.at[srcv.at[j + 2]], bufa, sema)

            pltpu.make_async_copy(h_hbm.at[srcv.at[j + 1]], bufb, semb).wait()
            scale(bufb, j + 1)
            pltpu.sync_copy(bufb, acc.at[dstv.at[j + 1]], add=True)

            @pl.when(j + 3 < NCH)
            def _():
                pltpu.async_copy(h_hbm.at[srcv.at[j + 3]], bufb, semb)

        plsc.subcore_barrier()

        pltpu.sync_copy(acc.at[pl.ds(zbase, ROWS_PER_TILE)],
                        out_hbm.at[cid].at[pl.ds(zbase, ROWS_PER_TILE)])

    return k(h, srcr, dstr, wr)


def _mm1_tc(x, w1):
    def body(x_ref, w_ref, o_ref):
        o_ref[...] = jnp.dot(x_ref[...], w_ref[...],
                             preferred_element_type=jnp.float32)

    return pl.pallas_call(
        body,
        out_shape=jax.ShapeDtypeStruct((N, D_H), jnp.float32),
    )(x, w1)


def _mm2_tc(p, w2):
    def body(p_ref, w_ref, o_ref):
        h = jnp.maximum(p_ref[0] + p_ref[1], 0.0)
        o_ref[...] = jnp.dot(h, w_ref[...],
                             preferred_element_type=jnp.float32)

    return pl.pallas_call(
        body,
        out_shape=jax.ShapeDtypeStruct((NPAD, D_OUT), jnp.float32),
    )(p, w2)


def _loss_tc(p2, label, maskf, w1):
    def body(p_ref, l_ref, m_ref, w1_ref, loss_ref, acc_ref):
        out = p_ref[0] + p_ref[1]
        lbl = l_ref[...]
        mx = jnp.max(out, axis=1, keepdims=True)
        ex = jnp.exp(out - mx)
        lse = jnp.log(jnp.sum(ex, axis=1, keepdims=True)) + mx
        logp = out - lse
        ce = -jnp.sum(lbl * logp, axis=1, keepdims=True)
        mf = m_ref[...]
        msum = jnp.sum(mf)

        iota = lax.broadcasted_iota(jnp.int32, out.shape, 1)
        big = jnp.int32(D_OUT)
        pred = jnp.min(jnp.where(out == mx, iota, big), axis=1, keepdims=True)
        lmx = jnp.max(lbl, axis=1, keepdims=True)
        lab = jnp.min(jnp.where(lbl == lmx, iota, big), axis=1, keepdims=True)
        correct = (pred == lab).astype(jnp.float32)

        wd = WEIGHT_DECAY * 0.5 * jnp.sum(w1_ref[...] * w1_ref[...])
        loss_ref[...] = (wd + jnp.sum(ce * mf) / msum).reshape(1, 1)
        acc_ref[...] = (jnp.sum(correct * mf) / msum).reshape(1, 1)

    return pl.pallas_call(
        body,
        out_shape=(jax.ShapeDtypeStruct((1, 1), jnp.float32),
                   jax.ShapeDtypeStruct((1, 1), jnp.float32)),
    )(p2, label, maskf, w1)


@jax.jit
def kernel(x, label, mask, edge_index, edge_weight, W1, W2):
    pad = EPAD - E
    src = jnp.concatenate([edge_index[0], jnp.zeros((pad,), jnp.int32)])
    dst = jnp.concatenate([edge_index[1], jnp.zeros((pad,), jnp.int32)])
    w = jnp.concatenate([edge_weight, jnp.zeros((pad,), jnp.float32)])
    srcr = src.reshape(32, NCH, C)
    dstr = dst.reshape(32, NCH, C)
    wr = w.reshape(32, EPT)

    h1 = _mm1_tc(x, W1)
    p1 = _spmm_sc(h1, srcr, dstr, wr, D_H)
    h2 = _mm2_tc(p1, W2)
    p2 = _spmm_sc(h2, srcr, dstr, wr, D_OUT)

    maskf = mask.astype(jnp.float32).reshape(N, 1)
    loss, acc = _loss_tc(p2[:, :N, :], label, maskf, W1)
    return (loss[0, 0], acc[0, 0])


# speedup vs baseline: 1.0138x; 1.0138x over previous
import functools

import jax
import jax.numpy as jnp
from jax import lax
from jax.experimental import pallas as pl
from jax.experimental.pallas import tpu as pltpu
from jax.experimental.pallas import tpu_sc as plsc

N = 10000
E = 320000
D_IN = 128
D_H = 64
D_OUT = 16
WEIGHT_DECAY = 0.0005

NPAD = 10240
C = 128
NCH = 80
ZB = 128
EPT = C * NCH
EPAD = 32 * EPT
ROWS_PER_TILE = NPAD // 16


def _spmm_sc(h, srcr, dstr, wr, d):
    mesh = plsc.VectorSubcoreMesh(core_axis_name="c", subcore_axis_name="s")

    @functools.partial(
        pl.kernel,
        out_type=jax.ShapeDtypeStruct((2, NPAD, d), jnp.float32),
        mesh=mesh,
        compiler_params=pltpu.CompilerParams(needs_layout_passes=False,
                                             use_tc_tiling_on_sc=False),
        scratch_types=[
            pltpu.VMEM((NCH, C), jnp.int32),
            pltpu.VMEM((NCH, C), jnp.int32),
            pltpu.VMEM((EPT,), jnp.float32),
            pltpu.VMEM((C, d), jnp.float32),
            pltpu.VMEM((C, d), jnp.float32),
            pltpu.VMEM((C, d), jnp.float32),
            pltpu.VMEM((C, d), jnp.float32),
            pltpu.VMEM((ZB, d), jnp.float32),
            pltpu.VMEM_SHARED((NPAD, d), jnp.float32),
            pltpu.SemaphoreType.DMA,
            pltpu.SemaphoreType.DMA,
            pltpu.SemaphoreType.DMA,
            pltpu.SemaphoreType.DMA,
        ],
    )
    def k(h_hbm, src_hbm, dst_hbm, w_hbm, out_hbm,
          srcv, dstv, wv, bufa, bufb, bufc, bufd, zbuf, acc,
          sema, semb, semc, semd):
        cid = lax.axis_index("c")
        sid = lax.axis_index("s")
        wid = cid * 16 + sid

        zero16 = jnp.zeros((16,), jnp.float32)
        iota0 = lax.iota(jnp.int32, 16)

        @pl.loop(0, ZB)
        def _(r):
            rv = jnp.full((16,), r, jnp.int32)
            for fb in range(d // 16):
                plsc.store_scatter(zbuf, [rv, iota0 + (fb * 16)], zero16)

        zbase = sid * ROWS_PER_TILE
        for i in range(ROWS_PER_TILE // ZB):
            pltpu.sync_copy(zbuf, acc.at[pl.ds(zbase + i * ZB, ZB)])

        pltpu.sync_copy(src_hbm.at[wid], srcv)
        pltpu.sync_copy(dst_hbm.at[wid], dstv)
        pltpu.sync_copy(w_hbm.at[wid], wv)

        plsc.subcore_barrier()

        dnums = lax.GatherDimensionNumbers(
            offset_dims=(), collapsed_slice_dims=(0,), start_index_map=(0,))
        idx16 = [jnp.full((16, 1), l, jnp.int32) for l in range(16)]

        def scale(buf, j):
            @pl.loop(0, C // 16)
            def _(g):
                wvec = wv.at[pl.ds(j * C + g * 16, 16)][...]
                for l in range(16):
                    s = lax.gather(
                        wvec, idx16[l], dnums, slice_sizes=(1,),
                        mode=lax.GatherScatterMode.PROMISE_IN_BOUNDS)
                    e = g * 16 + l
                    for fb in range(d // 16):
                        o = fb * 16
                        buf.at[e, pl.ds(o, 16)][...] = (
                            buf.at[e, pl.ds(o, 16)][...] * s)

        bufs = [bufa, bufb, bufc, bufd]
        sems = [sema, semb, semc, semd]
        for k4 in range(4):
            pltpu.async_copy(h_hbm.at[srcv.at[k4]], bufs[k4], sems[k4])

        @pl.loop(0, NCH, step=4)
        def _(j):
            for k4 in range(4):
                pltpu.make_async_copy(h_hbm.at[srcv.at[j + k4]],
                                      bufs[k4], sems[k4]).wait()
                scale(bufs[k4], j + k4)
                pltpu.sync_copy(bufs[k4], acc.at[dstv.at[j + k4]], add=True)

                @pl.when(j + k4 + 4 < NCH)
                def _():
                    pltpu.async_copy(h_hbm.at[srcv.at[j + k4 + 4]],
                                     bufs[k4], sems[k4])

        plsc.subcore_barrier()

        pltpu.sync_copy(acc.at[pl.ds(zbase, ROWS_PER_TILE)],
                        out_hbm.at[cid].at[pl.ds(zbase, ROWS_PER_TILE)])

    return k(h, srcr, dstr, wr)


def _mm1_tc(x, w1):
    def body(x_ref, w_ref, o_ref):
        o_ref[...] = jnp.dot(x_ref[...], w_ref[...],
                             preferred_element_type=jnp.float32)

    return pl.pallas_call(
        body,
        out_shape=jax.ShapeDtypeStruct((N, D_H), jnp.float32),
    )(x, w1)


def _mm2_tc(p, w2):
    def body(p_ref, w_ref, o_ref):
        h = jnp.maximum(p_ref[0] + p_ref[1], 0.0)
        o_ref[...] = jnp.dot(h, w_ref[...],
                             preferred_element_type=jnp.float32)

    return pl.pallas_call(
        body,
        out_shape=jax.ShapeDtypeStruct((NPAD, D_OUT), jnp.float32),
    )(p, w2)


def _loss_tc(p2, label, maskf, w1):
    def body(p_ref, l_ref, m_ref, w1_ref, loss_ref, acc_ref):
        out = p_ref[0] + p_ref[1]
        lbl = l_ref[...]
        mx = jnp.max(out, axis=1, keepdims=True)
        ex = jnp.exp(out - mx)
        lse = jnp.log(jnp.sum(ex, axis=1, keepdims=True)) + mx
        logp = out - lse
        ce = -jnp.sum(lbl * logp, axis=1, keepdims=True)
        mf = m_ref[...]
        msum = jnp.sum(mf)

        iota = lax.broadcasted_iota(jnp.int32, out.shape, 1)
        big = jnp.int32(D_OUT)
        pred = jnp.min(jnp.where(out == mx, iota, big), axis=1, keepdims=True)
        lmx = jnp.max(lbl, axis=1, keepdims=True)
        lab = jnp.min(jnp.where(lbl == lmx, iota, big), axis=1, keepdims=True)
        correct = (pred == lab).astype(jnp.float32)

        wd = WEIGHT_DECAY * 0.5 * jnp.sum(w1_ref[...] * w1_ref[...])
        loss_ref[...] = (wd + jnp.sum(ce * mf) / msum).reshape(1, 1)
        acc_ref[...] = (jnp.sum(correct * mf) / msum).reshape(1, 1)

    return pl.pallas_call(
        body,
        out_shape=(jax.ShapeDtypeStruct((1, 1), jnp.float32),
                   jax.ShapeDtypeStruct((1, 1), jnp.float32)),
    )(p2, label, maskf, w1)


@jax.jit
def kernel(x, label, mask, edge_index, edge_weight, W1, W2):
    pad = EPAD - E
    src = jnp.concatenate([edge_index[0], jnp.zeros((pad,), jnp.int32)])
    dst = jnp.concatenate([edge_index[1], jnp.zeros((pad,), jnp.int32)])
    w = jnp.concatenate([edge_weight, jnp.zeros((pad,), jnp.float32)])
    srcr = src.reshape(32, NCH, C)
    dstr = dst.reshape(32, NCH, C)
    wr = w.reshape(32, EPT)

    h1 = _mm1_tc(x, W1)
    p1 = _spmm_sc(h1, srcr, dstr, wr, D_H)
    h2 = _mm2_tc(p1, W2)
    p2 = _spmm_sc(h2, srcr, dstr, wr, D_OUT)

    maskf = mask.astype(jnp.float32).reshape(N, 1)
    loss, acc = _loss_tc(p2[:, :N, :], label, maskf, W1)
    return (loss[0, 0], acc[0, 0])


# speedup vs baseline: 1.2358x; 1.2189x over previous
import functools

import jax
import jax.numpy as jnp
from jax import lax
from jax.experimental import pallas as pl
from jax.experimental.pallas import tpu as pltpu
from jax.experimental.pallas import tpu_sc as plsc

N = 10000
E = 320000
D_IN = 128
D_H = 64
D_OUT = 16
WEIGHT_DECAY = 0.0005

NPAD = 10240
C = 128
NCH = 80
ZB = 16
EPT = C * NCH
EPAD = 32 * EPT
ROWS_PER_TILE = NPAD // 16


def _spmm_sc(h, srcr, dstr, wr, d):
    mesh = plsc.VectorSubcoreMesh(core_axis_name="c", subcore_axis_name="s")

    @functools.partial(
        pl.kernel,
        out_type=jax.ShapeDtypeStruct((2, NPAD, d), jnp.float32),
        mesh=mesh,
        compiler_params=pltpu.CompilerParams(needs_layout_passes=False,
                                             use_tc_tiling_on_sc=False),
        scratch_types=[
            pltpu.VMEM((NCH, C), jnp.int32),
            pltpu.VMEM((NCH, C), jnp.int32),
            pltpu.VMEM((EPT,), jnp.float32),
            pltpu.VMEM((C, d), jnp.float32),
            pltpu.VMEM((C, d), jnp.float32),
            pltpu.VMEM((ZB, d), jnp.float32),
            pltpu.VMEM_SHARED((NPAD, d), jnp.float32),
            pltpu.VMEM_SHARED((NPAD, d), jnp.float32),
            pltpu.SemaphoreType.DMA,
            pltpu.SemaphoreType.DMA,
        ],
    )
    def k(h_hbm, src_hbm, dst_hbm, w_hbm, out_hbm,
          srcv, dstv, wv, bufa, bufb, zbuf, acc, hstage,
          sema, semb):
        cid = lax.axis_index("c")
        sid = lax.axis_index("s")
        wid = cid * 16 + sid

        zero16 = jnp.zeros((16,), jnp.float32)
        iota0 = lax.iota(jnp.int32, 16)

        @pl.loop(0, ZB)
        def _(r):
            rv = jnp.full((16,), r, jnp.int32)
            for fb in range(d // 16):
                plsc.store_scatter(zbuf, [rv, iota0 + (fb * 16)], zero16)

        zbase = sid * ROWS_PER_TILE
        for i in range(ROWS_PER_TILE // ZB):
            pltpu.sync_copy(zbuf, acc.at[pl.ds(zbase + i * ZB, ZB)])

        pltpu.sync_copy(src_hbm.at[wid], srcv)
        pltpu.sync_copy(dst_hbm.at[wid], dstv)
        pltpu.sync_copy(w_hbm.at[wid], wv)

        n = h_hbm.shape[0]
        rpt = n // 16
        pltpu.sync_copy(h_hbm.at[pl.ds(sid * rpt, rpt)],
                        hstage.at[pl.ds(sid * rpt, rpt)])

        plsc.subcore_barrier()

        dnums = lax.GatherDimensionNumbers(
            offset_dims=(), collapsed_slice_dims=(0,), start_index_map=(0,))
        idx16 = [jnp.full((16, 1), l, jnp.int32) for l in range(16)]

        def scale(buf, j):
            @pl.loop(0, C // 16)
            def _(g):
                wvec = wv.at[pl.ds(j * C + g * 16, 16)][...]
                for l in range(16):
                    s = lax.gather(
                        wvec, idx16[l], dnums, slice_sizes=(1,),
                        mode=lax.GatherScatterMode.PROMISE_IN_BOUNDS)
                    e = g * 16 + l
                    for fb in range(d // 16):
                        o = fb * 16
                        buf.at[e, pl.ds(o, 16)][...] = (
                            buf.at[e, pl.ds(o, 16)][...] * s)

        pltpu.async_copy(hstage.at[srcv.at[0]], bufa, sema)
        pltpu.async_copy(hstage.at[srcv.at[1]], bufb, semb)

        @pl.loop(0, NCH, step=2)
        def _(j):
            pltpu.make_async_copy(hstage.at[srcv.at[j]], bufa, sema).wait()
            scale(bufa, j)
            pltpu.sync_copy(bufa, acc.at[dstv.at[j]], add=True)

            @pl.when(j + 2 < NCH)
            def _():
                pltpu.async_copy(hstage.at[srcv.at[j + 2]], bufa, sema)

            pltpu.make_async_copy(hstage.at[srcv.at[j + 1]], bufb, semb).wait()
            scale(bufb, j + 1)
            pltpu.sync_copy(bufb, acc.at[dstv.at[j + 1]], add=True)

            @pl.when(j + 3 < NCH)
            def _():
                pltpu.async_copy(hstage.at[srcv.at[j + 3]], bufb, semb)

        plsc.subcore_barrier()

        pltpu.sync_copy(acc.at[pl.ds(zbase, ROWS_PER_TILE)],
                        out_hbm.at[cid].at[pl.ds(zbase, ROWS_PER_TILE)])

    return k(h, srcr, dstr, wr)


def _mm1_tc(x, w1):
    def body(x_ref, w_ref, o_ref):
        o_ref[...] = jnp.dot(x_ref[...], w_ref[...],
                             preferred_element_type=jnp.float32)

    return pl.pallas_call(
        body,
        out_shape=jax.ShapeDtypeStruct((N, D_H), jnp.float32),
    )(x, w1)


def _mm2_tc(p, w2):
    def body(p_ref, w_ref, o_ref):
        h = jnp.maximum(p_ref[0] + p_ref[1], 0.0)
        o_ref[...] = jnp.dot(h, w_ref[...],
                             preferred_element_type=jnp.float32)

    return pl.pallas_call(
        body,
        out_shape=jax.ShapeDtypeStruct((NPAD, D_OUT), jnp.float32),
    )(p, w2)


def _loss_tc(p2, label, maskf, w1):
    def body(p_ref, l_ref, m_ref, w1_ref, loss_ref, acc_ref):
        out = p_ref[0] + p_ref[1]
        lbl = l_ref[...]
        mx = jnp.max(out, axis=1, keepdims=True)
        ex = jnp.exp(out - mx)
        lse = jnp.log(jnp.sum(ex, axis=1, keepdims=True)) + mx
        logp = out - lse
        ce = -jnp.sum(lbl * logp, axis=1, keepdims=True)
        mf = m_ref[...]
        msum = jnp.sum(mf)

        iota = lax.broadcasted_iota(jnp.int32, out.shape, 1)
        big = jnp.int32(D_OUT)
        pred = jnp.min(jnp.where(out == mx, iota, big), axis=1, keepdims=True)
        lmx = jnp.max(lbl, axis=1, keepdims=True)
        lab = jnp.min(jnp.where(lbl == lmx, iota, big), axis=1, keepdims=True)
        correct = (pred == lab).astype(jnp.float32)

        wd = WEIGHT_DECAY * 0.5 * jnp.sum(w1_ref[...] * w1_ref[...])
        loss_ref[...] = (wd + jnp.sum(ce * mf) / msum).reshape(1, 1)
        acc_ref[...] = (jnp.sum(correct * mf) / msum).reshape(1, 1)

    return pl.pallas_call(
        body,
        out_shape=(jax.ShapeDtypeStruct((1, 1), jnp.float32),
                   jax.ShapeDtypeStruct((1, 1), jnp.float32)),
    )(p2, label, maskf, w1)


@jax.jit
def kernel(x, label, mask, edge_index, edge_weight, W1, W2):
    pad = EPAD - E
    src = jnp.concatenate([edge_index[0], jnp.zeros((pad,), jnp.int32)])
    dst = jnp.concatenate([edge_index[1], jnp.zeros((pad,), jnp.int32)])
    w = jnp.concatenate([edge_weight, jnp.zeros((pad,), jnp.float32)])
    srcr = src.reshape(32, NCH, C)
    dstr = dst.reshape(32, NCH, C)
    wr = w.reshape(32, EPT)

    h1 = _mm1_tc(x, W1)
    p1 = _spmm_sc(h1, srcr, dstr, wr, D_H)
    h2 = _mm2_tc(p1, W2)
    p2 = _spmm_sc(h2, srcr, dstr, wr, D_OUT)

    maskf = mask.astype(jnp.float32).reshape(N, 1)
    loss, acc = _loss_tc(p2[:, :N, :], label, maskf, W1)
    return (loss[0, 0], acc[0, 0])


# speedup vs baseline: 1.3553x; 1.0967x over previous
import functools

import jax
import jax.numpy as jnp
from jax import lax
from jax.experimental import pallas as pl
from jax.experimental.pallas import tpu as pltpu
from jax.experimental.pallas import tpu_sc as plsc

N = 10000
E = 320000
D_IN = 128
D_H = 64
D_OUT = 16
WEIGHT_DECAY = 0.0005

NPAD = 10240
C = 128
NCH = 80
ZB = 16
EPT = C * NCH
EPAD = 32 * EPT
ROWS_PER_TILE = NPAD // 16


def _spmm_sc(h, srcr, dstr, wr, d, c, nch):
    mesh = plsc.VectorSubcoreMesh(core_axis_name="c", subcore_axis_name="s")

    @functools.partial(
        pl.kernel,
        out_type=jax.ShapeDtypeStruct((2, NPAD, d), jnp.float32),
        mesh=mesh,
        compiler_params=pltpu.CompilerParams(needs_layout_passes=False,
                                             use_tc_tiling_on_sc=False),
        scratch_types=[
            pltpu.VMEM((nch, c), jnp.int32),
            pltpu.VMEM((nch, c), jnp.int32),
            pltpu.VMEM((EPT,), jnp.float32),
            pltpu.VMEM((c, d), jnp.float32),
            pltpu.VMEM((c, d), jnp.float32),
            pltpu.VMEM((c, d), jnp.float32),
            pltpu.VMEM((c, d), jnp.float32),
            pltpu.VMEM((ZB, d), jnp.float32),
            pltpu.VMEM_SHARED((NPAD, d), jnp.float32),
            pltpu.VMEM_SHARED((NPAD, d), jnp.float32),
            pltpu.SemaphoreType.DMA,
            pltpu.SemaphoreType.DMA,
            pltpu.SemaphoreType.DMA,
            pltpu.SemaphoreType.DMA,
            pltpu.SemaphoreType.DMA,
            pltpu.SemaphoreType.DMA,
            pltpu.SemaphoreType.DMA,
            pltpu.SemaphoreType.DMA,
        ],
    )
    def k(h_hbm, src_hbm, dst_hbm, w_hbm, out_hbm,
          srcv, dstv, wv, bufa, bufb, bufc, bufd, zbuf, acc, hstage,
          sema, semb, semc, semd, ssa, ssb, ssc, ssd):
        cid = lax.axis_index("c")
        sid = lax.axis_index("s")
        wid = cid * 16 + sid

        zero16 = jnp.zeros((16,), jnp.float32)
        iota0 = lax.iota(jnp.int32, 16)

        @pl.loop(0, ZB)
        def _(r):
            rv = jnp.full((16,), r, jnp.int32)
            for fb in range(d // 16):
                plsc.store_scatter(zbuf, [rv, iota0 + (fb * 16)], zero16)

        zbase = sid * ROWS_PER_TILE
        for i in range(ROWS_PER_TILE // ZB):
            pltpu.sync_copy(zbuf, acc.at[pl.ds(zbase + i * ZB, ZB)])

        pltpu.sync_copy(src_hbm.at[wid], srcv)
        pltpu.sync_copy(dst_hbm.at[wid], dstv)
        pltpu.sync_copy(w_hbm.at[wid], wv)

        n = h_hbm.shape[0]
        rpt = n // 16
        pltpu.sync_copy(h_hbm.at[pl.ds(sid * rpt, rpt)],
                        hstage.at[pl.ds(sid * rpt, rpt)])

        plsc.subcore_barrier()

        dnums = lax.GatherDimensionNumbers(
            offset_dims=(), collapsed_slice_dims=(0,), start_index_map=(0,))
        idx16 = [jnp.full((16, 1), l, jnp.int32) for l in range(16)]

        def scale(buf, j):
            @pl.loop(0, c // 16)
            def _(g):
                wvec = wv.at[pl.ds(j * c + g * 16, 16)][...]
                for l in range(16):
                    s = lax.gather(
                        wvec, idx16[l], dnums, slice_sizes=(1,),
                        mode=lax.GatherScatterMode.PROMISE_IN_BOUNDS)
                    e = g * 16 + l
                    for fb in range(d // 16):
                        o = fb * 16
                        buf.at[e, pl.ds(o, 16)][...] = (
                            buf.at[e, pl.ds(o, 16)][...] * s)

        pltpu.async_copy(hstage.at[srcv.at[0]], bufa, sema)
        pltpu.async_copy(hstage.at[srcv.at[1]], bufb, semb)
        pltpu.async_copy(hstage.at[srcv.at[2]], bufc, semc)
        pltpu.async_copy(hstage.at[srcv.at[3]], bufd, semd)

        @pl.loop(0, nch, step=4)
        def _(j):
            pltpu.make_async_copy(hstage.at[srcv.at[j]], bufa, sema).wait()
            scale(bufa, j)

            @pl.when(j > 0)
            def _():
                pltpu.make_async_copy(bufd, acc.at[dstv.at[j - 1]], ssd).wait()
                pltpu.async_copy(hstage.at[srcv.at[j + 3]], bufd, semd)

            pltpu.async_copy(bufa, acc.at[dstv.at[j]], ssa, add=True)

            pltpu.make_async_copy(hstage.at[srcv.at[j + 1]], bufb, semb).wait()
            scale(bufb, j + 1)
            pltpu.make_async_copy(bufa, acc.at[dstv.at[j]], ssa).wait()

            @pl.when(j + 4 < nch)
            def _():
                pltpu.async_copy(hstage.at[srcv.at[j + 4]], bufa, sema)

            pltpu.async_copy(bufb, acc.at[dstv.at[j + 1]], ssb, add=True)

            pltpu.make_async_copy(hstage.at[srcv.at[j + 2]], bufc, semc).wait()
            scale(bufc, j + 2)
            pltpu.make_async_copy(bufb, acc.at[dstv.at[j + 1]], ssb).wait()

            @pl.when(j + 5 < nch)
            def _():
                pltpu.async_copy(hstage.at[srcv.at[j + 5]], bufb, semb)

            pltpu.async_copy(bufc, acc.at[dstv.at[j + 2]], ssc, add=True)

            pltpu.make_async_copy(hstage.at[srcv.at[j + 3]], bufd, semd).wait()
            scale(bufd, j + 3)
            pltpu.make_async_copy(bufc, acc.at[dstv.at[j + 2]], ssc).wait()

            @pl.when(j + 6 < nch)
            def _():
                pltpu.async_copy(hstage.at[srcv.at[j + 6]], bufc, semc)

            pltpu.async_copy(bufd, acc.at[dstv.at[j + 3]], ssd, add=True)

        pltpu.make_async_copy(bufd, acc.at[dstv.at[nch - 1]], ssd).wait()

        plsc.subcore_barrier()

        pltpu.sync_copy(acc.at[pl.ds(zbase, ROWS_PER_TILE)],
                        out_hbm.at[cid].at[pl.ds(zbase, ROWS_PER_TILE)])

    return k(h, srcr, dstr, wr)


def _mm1_tc(x, w1):
    def body(x_ref, w_ref, o_ref):
        o_ref[...] = jnp.dot(x_ref[...], w_ref[...],
                             preferred_element_type=jnp.float32)

    return pl.pallas_call(
        body,
        out_shape=jax.ShapeDtypeStruct((N, D_H), jnp.float32),
    )(x, w1)


def _mm2_tc(p, w2):
    def body(p_ref, w_ref, o_ref):
        h = jnp.maximum(p_ref[0] + p_ref[1], 0.0)
        o_ref[...] = jnp.dot(h, w_ref[...],
                             preferred_element_type=jnp.float32)

    return pl.pallas_call(
        body,
        out_shape=jax.ShapeDtypeStruct((NPAD, D_OUT), jnp.float32),
    )(p, w2)


def _loss_tc(p2, label, maskf, w1):
    def body(p_ref, l_ref, m_ref, w1_ref, loss_ref, acc_ref):
        out = p_ref[0] + p_ref[1]
        lbl = l_ref[...]
        mx = jnp.max(out, axis=1, keepdims=True)
        ex = jnp.exp(out - mx)
        lse = jnp.log(jnp.sum(ex, axis=1, keepdims=True)) + mx
        logp = out - lse
        ce = -jnp.sum(lbl * logp, axis=1, keepdims=True)
        mf = m_ref[...]
        msum = jnp.sum(mf)

        iota = lax.broadcasted_iota(jnp.int32, out.shape, 1)
        big = jnp.int32(D_OUT)
        pred = jnp.min(jnp.where(out == mx, iota, big), axis=1, keepdims=True)
        lmx = jnp.max(lbl, axis=1, keepdims=True)
        lab = jnp.min(jnp.where(lbl == lmx, iota, big), axis=1, keepdims=True)
        correct = (pred == lab).astype(jnp.float32)

        wd = WEIGHT_DECAY * 0.5 * jnp.sum(w1_ref[...] * w1_ref[...])
        loss_ref[...] = (wd + jnp.sum(ce * mf) / msum).reshape(1, 1)
        acc_ref[...] = (jnp.sum(correct * mf) / msum).reshape(1, 1)

    return pl.pallas_call(
        body,
        out_shape=(jax.ShapeDtypeStruct((1, 1), jnp.float32),
                   jax.ShapeDtypeStruct((1, 1), jnp.float32)),
    )(p2, label, maskf, w1)


@jax.jit
def kernel(x, label, mask, edge_index, edge_weight, W1, W2):
    pad = EPAD - E
    src = jnp.concatenate([edge_index[0], jnp.zeros((pad,), jnp.int32)])
    dst = jnp.concatenate([edge_index[1], jnp.zeros((pad,), jnp.int32)])
    w = jnp.concatenate([edge_weight, jnp.zeros((pad,), jnp.float32)])
    wr = w.reshape(32, EPT)

    h1 = _mm1_tc(x, W1)
    p1 = _spmm_sc(h1, src.reshape(32, 160, 64), dst.reshape(32, 160, 64),
                  wr, D_H, 64, 160)
    h2 = _mm2_tc(p1, W2)
    p2 = _spmm_sc(h2, src.reshape(32, NCH, C), dst.reshape(32, NCH, C),
                  wr, D_OUT, C, NCH)

    maskf = mask.astype(jnp.float32).reshape(N, 1)
    loss, acc = _loss_tc(p2[:, :N, :], label, maskf, W1)
    return (loss[0, 0], acc[0, 0])
